# initial kernel scaffold (unmeasured)
import functools

import jax
import jax.numpy as jnp
from jax import lax
from jax.experimental import pallas as pl
from jax.experimental.pallas import tpu as pltpu

N_DEV = 32
SQ = 1024
SKV = 1024
H_LOC = 8
DH = 128
COLS = H_LOC * DH
BLK = 64
SCALE = 0.08838834764831843
ROWS = SQ // N_DEV


def _attn_body(x_ref, wq_ref, k_ref, v_ref, wo_ref, o_ref):
    h = pl.program_id(0)
    x2 = x_ref[0]
    q = jnp.dot(x2, wq_ref[...], preferred_element_type=jnp.float32)
    k = k_ref[0, :, 0, :]
    s = lax.dot_general(
        q, k, (((1,), (1,)), ((), ())), preferred_element_type=jnp.float32
    ) * SCALE
    qb = lax.broadcasted_iota(jnp.int32, (SQ, SKV), 0) // BLK
    kb = lax.broadcasted_iota(jnp.int32, (SQ, SKV), 1) // BLK
    s = jnp.where(kb <= qb, s, -1e9)
    m = jnp.max(s, axis=1, keepdims=True)
    w = jnp.exp(s - m)
    w = w / jnp.sum(w, axis=1, keepdims=True)
    ctx = jnp.dot(w, v_ref[0, :, 0, :], preferred_element_type=jnp.float32)
    contrib = jnp.dot(ctx, wo_ref[...], preferred_element_type=jnp.float32)

    @pl.when(h == 0)
    def _():
        o_ref[...] = contrib

    @pl.when(h != 0)
    def _():
        o_ref[...] = o_ref[...] + contrib


def _attn_partial(x, wq_l, k_ext, v_ext, wo_l):
    return pl.pallas_call(
        _attn_body,
        grid=(H_LOC,),
        in_specs=[
            pl.BlockSpec((1, SQ, SQ), lambda h: (0, 0, 0)),
            pl.BlockSpec((SQ, DH), lambda h: (0, h)),
            pl.BlockSpec((1, SKV, 1, DH), lambda h: (0, 0, h, 0)),
            pl.BlockSpec((1, SKV, 1, DH), lambda h: (0, 0, h, 0)),
            pl.BlockSpec((DH, SQ), lambda h: (h, 0)),
        ],
        out_specs=pl.BlockSpec((SQ, SQ), lambda h: (0, 0)),
        out_shape=jax.ShapeDtypeStruct((SQ, SQ), jnp.float32),
    )(x, wq_l, k_ext, v_ext, wo_l)


def _allreduce_body(p_ref, o_ref, rs_buf, send_sem, recv_sems):
    d = lax.axis_index("i")
    left = lax.rem(d - 1 + N_DEV, N_DEV)
    right = lax.rem(d + 1, N_DEV)

    barrier_sem = pltpu.get_barrier_semaphore()
    for nbr in (left, right):
        pl.semaphore_signal(
            barrier_sem, inc=1,
            device_id=(nbr,), device_id_type=pl.DeviceIdType.MESH,
        )
    pl.semaphore_wait(barrier_sem, 2)

    o_ref[...] = p_ref[...]

    for st in range(N_DEV - 1):
        sc = lax.rem(d - st + N_DEV, N_DEV)
        rc = lax.rem(d - st - 1 + N_DEV, N_DEV)
        rdma = pltpu.make_async_remote_copy(
            src_ref=o_ref.at[pl.ds(sc * ROWS, ROWS)],
            dst_ref=rs_buf.at[st],
            send_sem=send_sem,
            recv_sem=recv_sems.at[st],
            device_id=(right,),
            device_id_type=pl.DeviceIdType.MESH,
        )
        rdma.start()
        rdma.wait()
        o_ref[pl.ds(rc * ROWS, ROWS), :] += rs_buf[st]

    for nbr in (left, right):
        pl.semaphore_signal(
            barrier_sem, inc=1,
            device_id=(nbr,), device_id_type=pl.DeviceIdType.MESH,
        )
    pl.semaphore_wait(barrier_sem, 2)

    for st in range(N_DEV - 1):
        sc = lax.rem(d + 1 - st + N_DEV, N_DEV)
        rdma = pltpu.make_async_remote_copy(
            src_ref=o_ref.at[pl.ds(sc * ROWS, ROWS)],
            dst_ref=o_ref.at[pl.ds(sc * ROWS, ROWS)],
            send_sem=send_sem,
            recv_sem=recv_sems.at[st],
            device_id=(right,),
            device_id_type=pl.DeviceIdType.MESH,
        )
        rdma.start()
        rdma.wait()


def _allreduce(p):
    return pl.pallas_call(
        _allreduce_body,
        in_specs=[pl.BlockSpec(memory_space=pltpu.VMEM)],
        out_specs=pl.BlockSpec(memory_space=pltpu.VMEM),
        out_shape=jax.ShapeDtypeStruct((SQ, SQ), jnp.float32),
        scratch_shapes=[
            pltpu.VMEM((N_DEV - 1, ROWS, SQ), jnp.float32),
            pltpu.SemaphoreType.DMA,
            pltpu.SemaphoreType.DMA((N_DEV - 1,)),
        ],
        compiler_params=pltpu.CompilerParams(collective_id=0),
    )(p)


def kernel(x, Wq, K_ext, V_ext, Wo):
    d = lax.axis_index("i")
    wq_l = lax.dynamic_slice(Wq, (0, d * COLS), (SQ, COLS))
    wo_l = lax.dynamic_slice(Wo, (d * COLS, 0), (COLS, SQ))
    partial = _attn_partial(x, wq_l, K_ext, V_ext, wo_l)
    out = _allreduce(partial)
    return out[None]


# baseline (device time: 263888 ns/iter reference)
import functools

import jax
import jax.numpy as jnp
from jax import lax
from jax.experimental import pallas as pl
from jax.experimental.pallas import tpu as pltpu

N_DEV = 32
SQ = 1024
SKV = 1024
H_LOC = 8
DH = 128
COLS = H_LOC * DH
BLK = 64
SCALE = 0.08838834764831843
ROWS = SQ // N_DEV


def _attn_body(x_ref, wq_ref, k_ref, v_ref, wo_ref, o_ref):
    h = pl.program_id(0)
    x2 = x_ref[0]
    q = jnp.dot(x2, wq_ref[...], preferred_element_type=jnp.float32)
    k = k_ref[...]
    s = lax.dot_general(
        q, k, (((1,), (1,)), ((), ())), preferred_element_type=jnp.float32
    ) * SCALE
    qb = lax.broadcasted_iota(jnp.int32, (SQ, SKV), 0) // BLK
    kb = lax.broadcasted_iota(jnp.int32, (SQ, SKV), 1) // BLK
    s = jnp.where(kb <= qb, s, -1e9)
    m = jnp.max(s, axis=1, keepdims=True)
    w = jnp.exp(s - m)
    w = w / jnp.sum(w, axis=1, keepdims=True)
    ctx = jnp.dot(w, v_ref[...], preferred_element_type=jnp.float32)
    contrib = jnp.dot(ctx, wo_ref[...], preferred_element_type=jnp.float32)

    @pl.when(h == 0)
    def _():
        o_ref[...] = contrib

    @pl.when(h != 0)
    def _():
        o_ref[...] = o_ref[...] + contrib


def _attn_partial(x, wq_l, k_ext, v_ext, wo_l):
    return pl.pallas_call(
        _attn_body,
        grid=(H_LOC,),
        in_specs=[
            pl.BlockSpec((1, SQ, SQ), lambda h: (0, 0, 0)),
            pl.BlockSpec((SQ, DH), lambda h: (0, h)),
            pl.BlockSpec((SKV, DH), lambda h: (0, h)),
            pl.BlockSpec((SKV, DH), lambda h: (0, h)),
            pl.BlockSpec((DH, SQ), lambda h: (h, 0)),
        ],
        out_specs=pl.BlockSpec((SQ, SQ), lambda h: (0, 0)),
        out_shape=jax.ShapeDtypeStruct((SQ, SQ), jnp.float32),
    )(x, wq_l, k_ext, v_ext, wo_l)


def _allreduce_body(p_ref, o_ref, rs_buf, send_sem, recv_sems):
    d = lax.axis_index("i")
    left = lax.rem(d - 1 + N_DEV, N_DEV)
    right = lax.rem(d + 1, N_DEV)

    barrier_sem = pltpu.get_barrier_semaphore()
    for nbr in (left, right):
        pl.semaphore_signal(
            barrier_sem, inc=1,
            device_id=(nbr,), device_id_type=pl.DeviceIdType.MESH,
        )
    pl.semaphore_wait(barrier_sem, 2)

    o_ref[...] = p_ref[...]

    for st in range(N_DEV - 1):
        sc = lax.rem(d - st + N_DEV, N_DEV)
        rc = lax.rem(d - st - 1 + N_DEV, N_DEV)
        rdma = pltpu.make_async_remote_copy(
            src_ref=o_ref.at[pl.ds(sc * ROWS, ROWS)],
            dst_ref=rs_buf.at[st],
            send_sem=send_sem,
            recv_sem=recv_sems.at[st],
            device_id=(right,),
            device_id_type=pl.DeviceIdType.MESH,
        )
        rdma.start()
        rdma.wait()
        o_ref[pl.ds(rc * ROWS, ROWS), :] += rs_buf[st]

    for nbr in (left, right):
        pl.semaphore_signal(
            barrier_sem, inc=1,
            device_id=(nbr,), device_id_type=pl.DeviceIdType.MESH,
        )
    pl.semaphore_wait(barrier_sem, 2)

    for st in range(N_DEV - 1):
        sc = lax.rem(d + 1 - st + N_DEV, N_DEV)
        rdma = pltpu.make_async_remote_copy(
            src_ref=o_ref.at[pl.ds(sc * ROWS, ROWS)],
            dst_ref=o_ref.at[pl.ds(sc * ROWS, ROWS)],
            send_sem=send_sem,
            recv_sem=recv_sems.at[st],
            device_id=(right,),
            device_id_type=pl.DeviceIdType.MESH,
        )
        rdma.start()
        rdma.wait()


def _allreduce(p):
    return pl.pallas_call(
        _allreduce_body,
        in_specs=[pl.BlockSpec(memory_space=pltpu.VMEM)],
        out_specs=pl.BlockSpec(memory_space=pltpu.VMEM),
        out_shape=jax.ShapeDtypeStruct((SQ, SQ), jnp.float32),
        scratch_shapes=[
            pltpu.VMEM((N_DEV - 1, ROWS, SQ), jnp.float32),
            pltpu.SemaphoreType.DMA,
            pltpu.SemaphoreType.DMA((N_DEV - 1,)),
        ],
        compiler_params=pltpu.CompilerParams(collective_id=0),
    )(p)


def kernel(x, Wq, K_ext, V_ext, Wo):
    d = lax.axis_index("i")
    wq_l = lax.dynamic_slice(Wq, (0, d * COLS), (SQ, COLS))
    wo_l = lax.dynamic_slice(Wo, (d * COLS, 0), (COLS, SQ))
    k2 = K_ext.reshape(SKV, H_LOC * DH)
    v2 = V_ext.reshape(SKV, H_LOC * DH)
    partial = _attn_partial(x, wq_l, k2, v2, wo_l)
    out = _allreduce(partial)
    return out[None]
